# baseline (speedup 1.0000x reference)
import jax
import jax.numpy as jnp
from jax import lax
from jax.experimental import pallas as pl
from jax.experimental.pallas import tpu as pltpu

_S = 30.0
_M = 0.35
_SUB = 8
_C1 = _S * 1.4426950408889634


def _stripe_body(n_rows, n_cls, n_blocks, bs, xa_ref, xb_ref, lblg_ref,
                 lblm_ref, out_ref, m_ref, s_ref, tg_ref):
    i = pl.program_id(0)
    half = bs // 2
    nh = half // _SUB
    ns = bs // _SUB
    tail_rows = n_cls - (n_blocks - 1) * bs
    na_tail = min(tail_rows, half) // _SUB
    nb_tail = max(tail_rows - half, 0) // _SUB

    @pl.when(i == 0)
    def _init():
        m_ref[...] = jnp.full_like(m_ref, -jnp.inf)
        s_ref[...] = jnp.zeros_like(s_ref)
        tg_ref[...] = jnp.zeros_like(tg_ref)

    lblg = lblg_ref[...]

    def update(na, nb):
        m_old = m_ref[...]
        bm = m_old
        tg = tg_ref[...]
        for ref, base, n_sl in ((xa_ref, 0, na), (xb_ref, nh, nb)):
            for k in range(n_sl):
                ch = ref[k * _SUB:(k + 1) * _SUB, :]
                bm = jnp.maximum(bm, ch)
                tg = jnp.where(lblg == i * ns + base + k, ch, tg)
        tg_ref[...] = tg
        acc = s_ref[...] * jnp.exp2(_C1 * (m_old - bm))
        for ref, n_sl in ((xa_ref, na), (xb_ref, nb)):
            for k in range(n_sl):
                ch = ref[k * _SUB:(k + 1) * _SUB, :]
                acc = acc + jnp.exp2(_C1 * (ch - bm))
        s_ref[...] = acc
        m_ref[...] = bm

    @pl.when(i < n_blocks - 1)
    def _main():
        update(nh, nh)

    @pl.when(i == n_blocks - 1)
    def _tail():
        update(na_tail, nb_tail)

        m8 = m_ref[...]
        mrow = jnp.max(m8, axis=0, keepdims=True)
        srow = jnp.sum(s_ref[...] * jnp.exp2(_C1 * (m8 - mrow)),
                       axis=0, keepdims=True)
        sub = lax.broadcasted_iota(jnp.int32, m8.shape, 0)
        t = jnp.sum(jnp.where(sub == lblm_ref[...], tg_ref[...], 0.0),
                    axis=0, keepdims=True)
        e1 = jnp.exp(_S * (t - mrow))
        e2 = jnp.exp(_S * (t - _M - mrow))
        s_corr = jnp.maximum(srow - e1, 0.0) + e2
        nll = jnp.log(s_corr) + _S * mrow - _S * (t - _M)
        out_ref[...] = jnp.sum(nll, axis=(0, 1), keepdims=True) / n_rows


@jax.jit
def kernel(input, label):
    n_rows, n_cls = input.shape
    xt = input.T
    lbl = label.astype(jnp.int32)
    lblg = jnp.broadcast_to((lbl // _SUB)[None, :], (_SUB, n_rows))
    lblm = jnp.broadcast_to((lbl % _SUB)[None, :], (_SUB, n_rows))

    bs = 2048
    n_blocks = pl.cdiv(n_cls, bs)
    body = lambda *refs: _stripe_body(n_rows, n_cls, n_blocks, bs, *refs)
    out = pl.pallas_call(
        body,
        grid=(n_blocks,),
        in_specs=[
            pl.BlockSpec((bs // 2, n_rows), lambda i: (2 * i, 0)),
            pl.BlockSpec((bs // 2, n_rows), lambda i: (2 * i + 1, 0)),
            pl.BlockSpec((_SUB, n_rows), lambda i: (0, 0)),
            pl.BlockSpec((_SUB, n_rows), lambda i: (0, 0)),
        ],
        out_specs=pl.BlockSpec((1, 1), lambda i: (0, 0)),
        out_shape=jax.ShapeDtypeStruct((1, 1), jnp.float32),
        scratch_shapes=[
            pltpu.VMEM((_SUB, n_rows), jnp.float32),
            pltpu.VMEM((_SUB, n_rows), jnp.float32),
            pltpu.VMEM((_SUB, n_rows), jnp.float32),
        ],
    )(xt, xt, lblg, lblm)
    return out[0, 0]


# speedup vs baseline: 1.0009x; 1.0009x over previous
import jax
import jax.numpy as jnp
from jax import lax
from jax.experimental import pallas as pl
from jax.experimental.pallas import tpu as pltpu

_S = 30.0
_M = 0.35
_SUB = 8
_C1 = _S * 1.4426950408889634


def _stripe_body(n_rows, n_cls, n_blocks, bs, xt_ref, lblg_ref, lblm_ref,
                 out_ref, m_ref, s_ref, tg_ref):
    i = pl.program_id(0)
    ns = bs // _SUB
    ns_tail = (n_cls - (n_blocks - 1) * bs) // _SUB

    @pl.when(i == 0)
    def _init():
        m_ref[...] = jnp.full_like(m_ref, -jnp.inf)
        s_ref[...] = jnp.zeros_like(s_ref)
        tg_ref[...] = jnp.zeros_like(tg_ref)

    lblg = lblg_ref[...]

    def update(n_slices):
        m_old = m_ref[...]
        bm = m_old
        tg = tg_ref[...]
        for k in range(n_slices):
            ch = xt_ref[k * _SUB:(k + 1) * _SUB, :]
            bm = jnp.maximum(bm, ch)
            tg = jnp.where(lblg == i * ns + k, ch, tg)
        tg_ref[...] = tg
        acc = s_ref[...] * jnp.exp2(_C1 * (m_old - bm))
        for k in range(n_slices):
            ch = xt_ref[k * _SUB:(k + 1) * _SUB, :]
            acc = acc + jnp.exp2(_C1 * (ch - bm))
        s_ref[...] = acc
        m_ref[...] = bm

    @pl.when(i < n_blocks - 1)
    def _main():
        update(ns)

    @pl.when(i == n_blocks - 1)
    def _tail():
        update(ns_tail)

        m8 = m_ref[...]
        mrow = jnp.max(m8, axis=0, keepdims=True)
        srow = jnp.sum(s_ref[...] * jnp.exp2(_C1 * (m8 - mrow)),
                       axis=0, keepdims=True)
        sub = lax.broadcasted_iota(jnp.int32, m8.shape, 0)
        t = jnp.sum(jnp.where(sub == lblm_ref[...], tg_ref[...], 0.0),
                    axis=0, keepdims=True)
        e1 = jnp.exp(_S * (t - mrow))
        e2 = jnp.exp(_S * (t - _M - mrow))
        s_corr = jnp.maximum(srow - e1, 0.0) + e2
        nll = jnp.log(s_corr) + _S * mrow - _S * (t - _M)
        out_ref[...] = jnp.sum(nll, axis=(0, 1), keepdims=True) / n_rows


@jax.jit
def kernel(input, label):
    n_rows, n_cls = input.shape
    xt = input.T
    lbl = label.astype(jnp.int32)
    lblg = jnp.broadcast_to((lbl // _SUB)[None, :], (_SUB, n_rows))
    lblm = jnp.broadcast_to((lbl % _SUB)[None, :], (_SUB, n_rows))

    bs = 2048
    n_blocks = pl.cdiv(n_cls, bs)
    body = lambda *refs: _stripe_body(n_rows, n_cls, n_blocks, bs, *refs)
    out = pl.pallas_call(
        body,
        grid=(n_blocks,),
        in_specs=[
            pl.BlockSpec((bs, n_rows), lambda i: (i, 0)),
            pl.BlockSpec((_SUB, n_rows), lambda i: (0, 0)),
            pl.BlockSpec((_SUB, n_rows), lambda i: (0, 0)),
        ],
        out_specs=pl.BlockSpec((1, 1), lambda i: (0, 0)),
        out_shape=jax.ShapeDtypeStruct((1, 1), jnp.float32),
        scratch_shapes=[
            pltpu.VMEM((_SUB, n_rows), jnp.float32),
            pltpu.VMEM((_SUB, n_rows), jnp.float32),
            pltpu.VMEM((_SUB, n_rows), jnp.float32),
        ],
    )(xt, lblg, lblm)
    return out[0, 0]
